# baseline (speedup 1.0000x reference)
import functools

import jax
import jax.numpy as jnp
from jax import lax
from jax.experimental import pallas as pl
from jax.experimental.pallas import tpu as pltpu
from jax.experimental.pallas import tpu_sc as plsc

NC = 2
NS = 16
NW = NC * NS
L = 16

LN2 = 0.6931471805599453
CHUNK_W = 128


def _sqrt_vec(x):
    x = jnp.maximum(x, jnp.float32(1e-30))
    i = plsc.bitcast(x, jnp.int32)
    i = jnp.int32(0x5F3759DF) - (i >> 1)
    y = plsc.bitcast(i, jnp.float32)
    for _ in range(3):
        y = y * (jnp.float32(1.5) - jnp.float32(0.5) * x * y * y)
    return x * y


def _make_kernel(B, F):
    SPW = B // NW
    CHUNK = CHUNK_W
    NCHUNK = SPW // CHUNK
    GROUPS = CHUNK // L
    DEPTH = 2

    mesh = plsc.VectorSubcoreMesh(core_axis_name="c", subcore_axis_name="s")

    @functools.partial(
        pl.kernel,
        out_type=jax.ShapeDtypeStruct((NW, L), jnp.float32),
        mesh=mesh,
        compiler_params=pltpu.CompilerParams(needs_layout_passes=False),
        scratch_types=[
            pltpu.VMEM((NCHUNK, CHUNK), jnp.int32),
            [pltpu.VMEM((CHUNK, F), jnp.float32)] * DEPTH,
            [pltpu.VMEM((CHUNK, F), jnp.float32)] * DEPTH,
            pltpu.VMEM((L,), jnp.float32),
            pltpu.VMEM((NCHUNK * GROUPS, L), jnp.float32),
            pltpu.VMEM((2 * GROUPS, L, L), jnp.float32),
            [pltpu.SemaphoreType.DMA] * DEPTH,
            [pltpu.SemaphoreType.DMA] * DEPTH,
        ],
    )
    def closs_kernel(xs_hbm, ys_hbm, center_hbm, out_hbm,
                     idx_v, xbuf, rbuf, acc_v, res_v, tp_v, sg, sx):
        cid = lax.axis_index("c")
        sid = lax.axis_index("s")
        wid = sid * NC + cid

        pltpu.sync_copy(ys_hbm.at[pl.ds(wid * NCHUNK, NCHUNK)], idx_v)

        def start(k):
            b = k % DEPTH
            gd = pltpu.async_copy(center_hbm.at[idx_v.at[k]], rbuf[b], sg[b])
            xd = pltpu.async_copy(
                xs_hbm.at[pl.ds(wid * SPW + k * CHUNK, CHUNK)], xbuf[b], sx[b])
            return gd, xd

        pending = {k: start(k) for k in range(DEPTH - 1)}
        lane = lax.iota(jnp.int32, L)

        for k in range(NCHUNK):
            b = k % DEPTH
            if k + DEPTH - 1 < NCHUNK:
                pending[k + DEPTH - 1] = start(k + DEPTH - 1)
            gd, xd = pending.pop(k)
            gd.wait()
            xd.wait()
            xs_v, rows_v = xbuf[b], rbuf[b]

            @plsc.parallel_loop(0, GROUPS)
            def gbody(g, xs_v=xs_v, rows_v=rows_v, k=k):
                base = g * jnp.int32(L)
                mask15 = jnp.full((L,), L - 1, jnp.int32)
                for i in range(L):
                    s_row = base + jnp.int32(i)
                    ts = []
                    for j in range(F // L):
                        xc = xs_v[s_row, pl.ds(j * L, L)]
                        cc = rows_v[s_row, pl.ds(j * L, L)]
                        ts.append(jnp.exp((xc - cc) * jnp.float32(LN2)))
                    while len(ts) > 1:
                        ts = [a + b for a, b in zip(ts[::2], ts[1::2])]
                    rot = (lane + jnp.int32(i)) & mask15
                    plsc.store_scatter(
                        tp_v.at[g + jnp.int32((k % 2) * GROUPS)],
                        [jnp.full((L,), i, jnp.int32), rot], ts[0])
                cs = []
                for j in range(L):
                    colj = (lane + jnp.int32(j)) & mask15
                    cs.append(plsc.load_gather(
                        tp_v.at[g + jnp.int32((k % 2) * GROUPS)], [lane, colj]))
                while len(cs) > 1:
                    cs = [a + b for a, b in zip(cs[::2], cs[1::2])]
                res_v[g + jnp.int32(k * GROUPS)] = _sqrt_vec(cs[0])

        accs = [jnp.zeros((L,), jnp.float32) for _ in range(4)]
        for j in range(NCHUNK * GROUPS):
            accs[j % 4] = accs[j % 4] + res_v[j]
        acc_v[...] = (accs[0] + accs[1]) + (accs[2] + accs[3])
        pltpu.sync_copy(acc_v, out_hbm.at[wid])

    return closs_kernel


def kernel(xs, ys, center):
    B, F = xs.shape
    ys2d = ys.astype(jnp.int32).reshape(B // CHUNK_W, CHUNK_W)
    partials = _make_kernel(B, F)(xs, ys2d, center)
    return jnp.sum(partials)


# speedup vs baseline: 1.2784x; 1.2784x over previous
import functools

import jax
import jax.numpy as jnp
from jax import lax
from jax.experimental import pallas as pl
from jax.experimental.pallas import tpu as pltpu
from jax.experimental.pallas import tpu_sc as plsc

NC = 2
NS = 16
NW = NC * NS
L = 16

LN2 = 0.6931471805599453
CHUNK_W = 128


def _sqrt_vec(x):
    x = jnp.maximum(x, jnp.float32(1e-30))
    i = plsc.bitcast(x, jnp.int32)
    i = jnp.int32(0x5F3759DF) - (i >> 1)
    y = plsc.bitcast(i, jnp.float32)
    for _ in range(3):
        y = y * (jnp.float32(1.5) - jnp.float32(0.5) * x * y * y)
    return x * y


def _make_kernel(B, F):
    SPW = B // NW
    CHUNK = CHUNK_W
    NCHUNK = SPW // CHUNK
    GROUPS = CHUNK // L
    DEPTH = 2

    mesh = plsc.VectorSubcoreMesh(core_axis_name="c", subcore_axis_name="s")

    @functools.partial(
        pl.kernel,
        out_type=jax.ShapeDtypeStruct((NW, L), jnp.float32),
        mesh=mesh,
        compiler_params=pltpu.CompilerParams(needs_layout_passes=False),
        scratch_types=[
            pltpu.VMEM((NCHUNK, CHUNK), jnp.int32),
            [pltpu.VMEM((CHUNK, F), jnp.float32)] * DEPTH,
            [pltpu.VMEM((CHUNK, F), jnp.float32)] * DEPTH,
            pltpu.VMEM((L,), jnp.float32),
            pltpu.VMEM((NCHUNK * GROUPS, L), jnp.float32),
            pltpu.VMEM((2 * GROUPS, L, L), jnp.float32),
            [pltpu.SemaphoreType.DMA] * DEPTH,
            [pltpu.SemaphoreType.DMA] * DEPTH,
        ],
    )
    def closs_kernel(xs_hbm, ys_hbm, center_hbm, out_hbm,
                     idx_v, xbuf, rbuf, acc_v, res_v, tp_v, sg, sx):
        cid = lax.axis_index("c")
        sid = lax.axis_index("s")
        wid = sid * NC + cid

        pltpu.sync_copy(ys_hbm.at[pl.ds(wid * NCHUNK, NCHUNK)], idx_v)

        def start(k):
            b = k % DEPTH
            gd = pltpu.async_copy(center_hbm.at[idx_v.at[k]], rbuf[b], sg[b])
            xd = pltpu.async_copy(
                xs_hbm.at[pl.ds(wid * SPW + k * CHUNK, CHUNK)], xbuf[b], sx[b])
            return gd, xd

        pending = {k: start(k) for k in range(DEPTH - 1)}
        lane = lax.iota(jnp.int32, L)

        for k in range(NCHUNK):
            b = k % DEPTH
            if k + DEPTH - 1 < NCHUNK:
                pending[k + DEPTH - 1] = start(k + DEPTH - 1)
            gd, xd = pending.pop(k)
            gd.wait()
            xd.wait()
            xs_v, rows_v = xbuf[b], rbuf[b]

            @plsc.parallel_loop(0, GROUPS)
            def gbody(g, xs_v=xs_v, rows_v=rows_v, k=k):
                row = lane + g * jnp.int32(L)
                ss = [jnp.zeros((L,), jnp.float32) for _ in range(4)]
                col = lane
                fvec = jnp.full((L,), F, jnp.int32)
                one = jnp.full((L,), 1, jnp.int32)
                dpair = None
                for f in range(F):
                    xc = plsc.load_gather(xs_v, [row, col])
                    cc = plsc.load_gather(rows_v, [row, col])
                    d = (xc - cc) * jnp.float32(LN2)
                    if dpair is None:
                        dpair = d
                    else:
                        e = jnp.exp(plsc.pack(dpair, d,
                                              format=plsc.PackFormat.INTERLEAVED))
                        u0, u1 = plsc.unpack(e,
                                             format=plsc.PackFormat.INTERLEAVED)
                        ss[(f // 2) % 4] = ss[(f // 2) % 4] + (u0 + u1)
                        dpair = None
                    if f + 1 < F:
                        col = col + one
                        if f + 1 > F - L:
                            col = jnp.where(col >= fvec, col - fvec, col)
                s = (ss[0] + ss[1]) + (ss[2] + ss[3])
                res_v[g + jnp.int32(k * GROUPS)] = _sqrt_vec(s)

        accs = [jnp.zeros((L,), jnp.float32) for _ in range(4)]
        for j in range(NCHUNK * GROUPS):
            accs[j % 4] = accs[j % 4] + res_v[j]
        acc_v[...] = (accs[0] + accs[1]) + (accs[2] + accs[3])
        pltpu.sync_copy(acc_v, out_hbm.at[wid])

    return closs_kernel


def kernel(xs, ys, center):
    B, F = xs.shape
    ys2d = ys.astype(jnp.int32).reshape(B // CHUNK_W, CHUNK_W)
    partials = _make_kernel(B, F)(xs, ys2d, center)
    return jnp.sum(partials)
